# Initial kernel scaffold; baseline (speedup 1.0000x reference)
import jax
import jax.numpy as jnp
from jax.experimental import pallas as pl


def kernel(updates, mask):
    raise NotImplementedError("write your pallas kernel here")


# speedup vs baseline: 10.4779x; 10.4779x over previous
import functools

import jax
import jax.numpy as jnp
from jax import lax
from jax.experimental import pallas as pl
from jax.experimental.pallas import tpu as pltpu
from jax.experimental.pallas import tpu_sc as plsc

_B, _H, _W, _C = 4, 192, 192, 96
_UP = 2
_OUT_H, _OUT_W = _H * _UP, _W * _UP
_FLAT_OUT = _OUT_H * _OUT_W * _C
_N_IN = _H * _W * _C

_NC, _NS = 2, 16
_NSEG = 12
_SEG = _FLAT_OUT // _NSEG
_DUMP = 2048
_ROUNDS = (_B * _NSEG) // _NC

_CHUNK_ROWS = 64
_CHUNK = _CHUNK_ROWS * 128
_TILE_ELEMS = _N_IN // _NS
_NSTEP = _TILE_ELEMS // _CHUNK
_TILE_ROWS = _TILE_ELEMS // 128
_IN_ROWS_PER_BATCH = _N_IN // 128

_SLICE = _SEG // _NS
_ZCHUNK = _SLICE // 4


def _unpool_body(upd_hbm, mask_hbm, out_hbm, idx_v, upd_v, zero_v, seg_sh):
    c = lax.axis_index("c")
    s = lax.axis_index("s")

    def _zinit(i, carry):
        zero_v[pl.ds(i * 16, 16)] = jnp.zeros((16,), jnp.float32)
        return carry

    lax.fori_loop(0, _ZCHUNK // 16, _zinit, 0)

    def _round(r, carry):
        rr = r * _NC + c
        b = rr // _NSEG
        sg = rr % _NSEG
        seg_base = sg * _SEG

        for z in range(_SLICE // _ZCHUNK):
            pltpu.sync_copy(
                zero_v, seg_sh.at[pl.ds(s * _SLICE + z * _ZCHUNK, _ZCHUNK)]
            )
        plsc.subcore_barrier()

        ebase0 = b * _N_IN + s * _TILE_ELEMS

        def _step(w, carry2):
            ebase = ebase0 + w * _CHUNK
            pltpu.sync_copy(mask_hbm.at[pl.ds(ebase, _CHUNK)], idx_v)
            pltpu.sync_copy(upd_hbm.at[pl.ds(ebase, _CHUNK)], upd_v)

            def _remap(i, carry3):
                for j in range(8):
                    o = i * 128 + j * 16
                    iv = idx_v[pl.ds(o, 16)]
                    local = iv - seg_base
                    inseg = (local >= 0) & (local < _SEG)
                    dump = _SEG + (local & (_DUMP - 1))
                    idx_v[pl.ds(o, 16)] = jnp.where(inseg, local, dump)
                return carry3

            lax.fori_loop(0, _CHUNK // 128, _remap, 0)
            pltpu.sync_copy(upd_v, seg_sh.at[idx_v], add=True)
            return carry2

        lax.fori_loop(0, _NSTEP, _step, 0)
        plsc.subcore_barrier()

        out_base = b * _FLAT_OUT + seg_base + s * _SLICE
        pltpu.sync_copy(
            seg_sh.at[pl.ds(s * _SLICE, _SLICE)],
            out_hbm.at[pl.ds(out_base, _SLICE)],
        )
        return carry

    lax.fori_loop(0, _ROUNDS, _round, 0)


_unpool_sc = pl.kernel(
    _unpool_body,
    out_type=jax.ShapeDtypeStruct((_B * _FLAT_OUT,), jnp.float32),
    mesh=plsc.VectorSubcoreMesh(core_axis_name="c", subcore_axis_name="s"),
    scratch_types=[
        pltpu.VMEM((_CHUNK,), jnp.int32),
        pltpu.VMEM((_CHUNK,), jnp.float32),
        pltpu.VMEM((_ZCHUNK,), jnp.float32),
        pltpu.VMEM_SHARED((_SEG + _DUMP,), jnp.float32),
    ],
)


@jax.jit
def kernel(updates, mask):
    upd1 = updates.reshape(_B * _N_IN)
    mask1 = mask.reshape(_B * _N_IN)
    flat = _unpool_sc(upd1, mask1)
    return flat.reshape(_B, _OUT_H, _OUT_W, _C)
